# baseline (speedup 1.0000x reference)
import jax
import jax.numpy as jnp
from jax.experimental import pallas as pl
from jax.experimental.pallas import tpu as pltpu
from functools import partial


def _dkvmn_kernel(num_q, S, B, DK, DV,
                  vx_ref,
                  femb_ref,
                  wblk_ref,
                  mv03_ref,
                  fW1T_ref,
                  fb_ref, eb_ref, ab_ref, pW_ref,
                  pb_ref,
                  out_ref,
                  fslt,
                  wslt,
                  eslt, aslt, kfslt,
                  rslt,
                  mv_scr):
    mv_scr[...] = mv03_ref[...]

    def gather_step(s, par):
        for b in range(B):
            fslt[par, b] = femb_ref[vx_ref[b, s], 0]

    def project_step(par):
        kv = fslt[par]
        acts = jax.lax.dot_general(
            wblk_ref[...], kv, (((1,), (1,)), ((), ())),
            preferred_element_type=jnp.float32)
        logitT = acts[0:DV]
        m = jnp.max(logitT, axis=0, keepdims=True)
        ex = jnp.exp(logitT - m)
        w = ex / jnp.sum(ex, axis=0, keepdims=True)
        wslt[pl.ds(par * DV, DV)] = w[:, None, :]
        kfslt[par] = acts[DV:DV + DK]
        eslt[par] = jax.nn.sigmoid(acts[DV + DK:DV + 2 * DK] + eb_ref[...])
        aslt[par] = jnp.tanh(acts[DV + 2 * DK:DV + 3 * DK] + ab_ref[...])

    def head(par, srow):
        fT = jnp.tanh(jax.lax.dot_general(
            fW1T_ref[...], rslt[par], (((1,), (0,)), ((), ())),
            preferred_element_type=jnp.float32) + kfslt[par] + fb_ref[...])
        logit = jnp.sum(fT * pW_ref[...], axis=0, keepdims=True) + pb_ref[...]
        out_ref[pl.ds(srow, 1), :] = jax.nn.sigmoid(logit)

    gather_step(0, 0)
    gather_step(1, 1)
    project_step(0)

    def step(s, par, nxt):
        e3 = eslt[par]
        a3 = aslt[par]
        wbase = par * DV
        VC = 8
        rT = None
        for vc in range(0, DV, VC):
            Mc = mv_scr[vc:vc + VC]
            w3c = wslt[pl.ds(wbase + vc, VC)]
            Pc = Mc * w3c
            ps = jnp.sum(Pc, axis=0)
            rT = ps if rT is None else rT + ps
            mv_scr[vc:vc + VC] = Mc - Pc * e3[None] + w3c * a3[None]

        head(nxt, jnp.maximum(s - 1, 0))
        rslt[par] = rT

        project_step(nxt)

        sg = jnp.minimum(s + 2, S - 1)
        gather_step(sg, par)

    def body(s, _):
        par = jax.lax.rem(s, 2)
        nxt = jax.lax.rem(s + 1, 2)
        step(s, par, nxt)
        return 0

    jax.lax.fori_loop(0, S, body, 0)
    head((S - 1) % 2, S - 1)


def kernel(question_seq, correct_seq, k_emb, v_emb, Mk, Mv0, fW, fb, eW, eb,
           aW, ab, pW, pb):
    B, S = question_seq.shape
    num_q, DK = k_emb.shape
    DV = Mk.shape[0]

    qs = question_seq.astype(jnp.int32)
    vx = qs + num_q * correct_seq.astype(jnp.int32)
    femb = jnp.concatenate(
        [jnp.concatenate([k_emb, k_emb], axis=0), v_emb], axis=1)
    femb3 = femb.reshape(2 * num_q, 1, 2 * DK)
    mv03 = jnp.broadcast_to(Mv0[:, :, None], (DV, DK, B))
    fW1T = fW[:DK].T
    fW2T = fW[DK:].T
    eWT = eW.T
    aWT = aW.T
    zz = jnp.zeros((DK, DK), jnp.float32)
    zv = jnp.zeros((DV, DK), jnp.float32)
    wblk = jnp.concatenate([
        jnp.concatenate([Mk, zv], axis=1),
        jnp.concatenate([fW2T, zz], axis=1),
        jnp.concatenate([zz, eWT], axis=1),
        jnp.concatenate([zz, aWT], axis=1),
    ], axis=0)
    fb2 = fb.reshape(DK, 1)
    eb2 = eb.reshape(DK, 1)
    ab2 = ab.reshape(DK, 1)
    pW2 = pW.reshape(DK, 1)
    pb2 = pb.reshape(1, 1)

    out = pl.pallas_call(
        partial(_dkvmn_kernel, num_q, S, B, DK, DV),
        out_shape=jax.ShapeDtypeStruct((S, B), jnp.float32),
        grid=(1,),
        in_specs=[
            pl.BlockSpec(memory_space=pltpu.SMEM),
        ] + [pl.BlockSpec(memory_space=pltpu.VMEM)] * 9,
        out_specs=pl.BlockSpec((S, B), lambda i: (0, 0)),
        scratch_shapes=[
            pltpu.VMEM((2, B, 2 * DK), jnp.float32),
            pltpu.VMEM((2 * DV, 1, B), jnp.float32),
            pltpu.VMEM((2, DK, B), jnp.float32),
            pltpu.VMEM((2, DK, B), jnp.float32),
            pltpu.VMEM((2, DK, B), jnp.float32),
            pltpu.VMEM((2, DK, B), jnp.float32),
            pltpu.VMEM((DV, DK, B), jnp.float32),
        ],
        compiler_params=pltpu.CompilerParams(
            dimension_semantics=("arbitrary",),
            vmem_limit_bytes=48 * 1024 * 1024,
        ),
        name="dkvmn_fused",
    )(vx, femb3, wblk, mv03, fW1T, fb2, eb2, ab2, pW2, pb2)
    return out.T


# speedup vs baseline: 1.2676x; 1.2676x over previous
import jax
import jax.numpy as jnp
from jax.experimental import pallas as pl
from jax.experimental.pallas import tpu as pltpu
from functools import partial


def _dkvmn_kernel(num_q, S, B, DK, DV,
                  qs_ref, vx_ref,
                  kemb_ref, vemb_ref,
                  mk_ref,
                  mv03_ref,
                  fW1T_ref, fW2T_ref, eWT_ref, aWT_ref,
                  fb_ref, eb_ref, ab_ref, pW_ref,
                  pb_ref,
                  out_ref,
                  kslt, vslt,
                  wslt,
                  eslt, aslt, kfslt,
                  rslt,
                  mv_scr):
    mv_scr[...] = mv03_ref[...]

    def gather_step(s, par):
        for b in range(B):
            kslt[par, b] = kemb_ref[qs_ref[b, s], 0]
            vslt[par, b] = vemb_ref[vx_ref[b, s], 0]

    def project_step(par):
        k_t = kslt[par]
        v_t = vslt[par]
        logitT = jax.lax.dot_general(
            mk_ref[...], k_t, (((1,), (1,)), ((), ())),
            preferred_element_type=jnp.float32)
        m = jnp.max(logitT, axis=0, keepdims=True)
        ex = jnp.exp(logitT - m)
        w = ex / jnp.sum(ex, axis=0, keepdims=True)
        wslt[pl.ds(par * DV, DV)] = w[:, None, :]
        eslt[par] = jax.nn.sigmoid(jax.lax.dot_general(
            eWT_ref[...], v_t, (((1,), (1,)), ((), ())),
            preferred_element_type=jnp.float32) + eb_ref[...])
        aslt[par] = jnp.tanh(jax.lax.dot_general(
            aWT_ref[...], v_t, (((1,), (1,)), ((), ())),
            preferred_element_type=jnp.float32) + ab_ref[...])
        kfslt[par] = jax.lax.dot_general(
            fW2T_ref[...], k_t, (((1,), (1,)), ((), ())),
            preferred_element_type=jnp.float32)

    def head(par, srow):
        fT = jnp.tanh(jax.lax.dot_general(
            fW1T_ref[...], rslt[par], (((1,), (0,)), ((), ())),
            preferred_element_type=jnp.float32) + kfslt[par] + fb_ref[...])
        logit = jnp.sum(fT * pW_ref[...], axis=0, keepdims=True) + pb_ref[...]
        out_ref[pl.ds(srow, 1), :] = jax.nn.sigmoid(logit)

    gather_step(0, 0)
    gather_step(1, 1)
    project_step(0)

    def step(s, par, nxt):
        e3 = eslt[par]
        a3 = aslt[par]
        wbase = par * DV
        VC = 8
        rT = None
        for vc in range(0, DV, VC):
            Mc = mv_scr[vc:vc + VC]
            w3c = wslt[pl.ds(wbase + vc, VC)]
            Pc = Mc * w3c
            ps = jnp.sum(Pc, axis=0)
            rT = ps if rT is None else rT + ps
            mv_scr[vc:vc + VC] = Mc - Pc * e3[None] + w3c * a3[None]

        head(nxt, jnp.maximum(s - 1, 0))
        rslt[par] = rT

        project_step(nxt)

        sg = jnp.minimum(s + 2, S - 1)
        gather_step(sg, par)

    def body(s, _):
        par = jax.lax.rem(s, 2)
        nxt = jax.lax.rem(s + 1, 2)
        step(s, par, nxt)
        return 0

    jax.lax.fori_loop(0, S, body, 0)
    head((S - 1) % 2, S - 1)


def kernel(question_seq, correct_seq, k_emb, v_emb, Mk, Mv0, fW, fb, eW, eb,
           aW, ab, pW, pb):
    B, S = question_seq.shape
    num_q, DK = k_emb.shape
    DV = Mk.shape[0]

    qs = question_seq.astype(jnp.int32)
    vx = qs + num_q * correct_seq.astype(jnp.int32)
    kemb3 = k_emb.reshape(num_q, 1, DK)
    vemb3 = v_emb.reshape(2 * num_q, 1, DK)
    mv03 = jnp.broadcast_to(Mv0[:, :, None], (DV, DK, B))
    fW1T = fW[:DK].T
    fW2T = fW[DK:].T
    eWT = eW.T
    aWT = aW.T
    fb2 = fb.reshape(DK, 1)
    eb2 = eb.reshape(DK, 1)
    ab2 = ab.reshape(DK, 1)
    pW2 = pW.reshape(DK, 1)
    pb2 = pb.reshape(1, 1)

    out = pl.pallas_call(
        partial(_dkvmn_kernel, num_q, S, B, DK, DV),
        out_shape=jax.ShapeDtypeStruct((S, B), jnp.float32),
        grid=(1,),
        in_specs=[
            pl.BlockSpec(memory_space=pltpu.SMEM),
            pl.BlockSpec(memory_space=pltpu.SMEM),
        ] + [pl.BlockSpec(memory_space=pltpu.VMEM)] * 13,
        out_specs=pl.BlockSpec((S, B), lambda i: (0, 0)),
        scratch_shapes=[
            pltpu.VMEM((2, B, DK), jnp.float32),
            pltpu.VMEM((2, B, DK), jnp.float32),
            pltpu.VMEM((2 * DV, 1, B), jnp.float32),
            pltpu.VMEM((2, DK, B), jnp.float32),
            pltpu.VMEM((2, DK, B), jnp.float32),
            pltpu.VMEM((2, DK, B), jnp.float32),
            pltpu.VMEM((2, DK, B), jnp.float32),
            pltpu.VMEM((DV, DK, B), jnp.float32),
        ],
        compiler_params=pltpu.CompilerParams(
            dimension_semantics=("arbitrary",),
            vmem_limit_bytes=48 * 1024 * 1024,
        ),
        name="dkvmn_fused",
    )(qs, vx, kemb3, vemb3, Mk, mv03, fW1T, fW2T, eWT, aWT, fb2, eb2, ab2,
      pW2, pb2)
    return out.T
